# Initial kernel scaffold; baseline (speedup 1.0000x reference)
import jax
import jax.numpy as jnp
from jax.experimental import pallas as pl


def kernel(x_num, cand_x_num, W_lin, b_lin, W_K, b_K, Y_emb, W_T1, b_T1, W_T2, ln1_s, ln1_b, W_P1, b_P1, W_P2, b_P2, lnP_s, lnP_b, W_out, b_out, cand_y, context_size):
    raise NotImplementedError("write your pallas kernel here")


# speedup vs baseline: 1.0072x; 1.0072x over previous
import jax
import jax.numpy as jnp
from jax.experimental import pallas as pl

B = 1024
N = 32768
F = 256
D = 265
DI = 530
NC = 10
CTX = 96


def _enc_cand_body(cx_ref, wl_ref, bl_ref, wk_ref, bk_ref, ki_ref):
    xe = jnp.dot(cx_ref[...], wl_ref[...], preferred_element_type=jnp.float32)
    xe = xe + bl_ref[...]
    ki = jnp.dot(xe, wk_ref[...], preferred_element_type=jnp.float32)
    ki_ref[...] = ki + bk_ref[...]


def _enc_query_body(x_ref, wl_ref, bl_ref, wk_ref, bk_ref, xe_ref, k_ref):
    xe = jnp.dot(x_ref[...], wl_ref[...], preferred_element_type=jnp.float32)
    xe = xe + bl_ref[...]
    xe_ref[...] = xe
    k = jnp.dot(xe, wk_ref[...], preferred_element_type=jnp.float32)
    k_ref[...] = k + bk_ref[...]


def _dist_body(k_ref, ki_ref, out_ref):
    k = k_ref[...]
    ki = ki_ref[...]
    kk = jnp.sum(k * k, axis=1, keepdims=True)
    kin = jnp.sum(ki * ki, axis=1, keepdims=True)
    prod = jax.lax.dot_general(
        k, ki, (((1,), (1,)), ((), ())),
        preferred_element_type=jnp.float32)
    out_ref[...] = kk - 2.0 * prod + kin.T


def _ln_in(x, s, b, eps=1e-5):
    m = jnp.mean(x, axis=-1, keepdims=True)
    v = jnp.mean((x - m) ** 2, axis=-1, keepdims=True)
    return (x - m) / jnp.sqrt(v + eps) * s + b


def _final_body(k_ref, xe_ref, ki_ref, ey_ref, ds_ref,
                wt1_ref, bt1_ref, wt2_ref,
                ln1s_ref, ln1b_ref, wp1_ref, bp1_ref, wp2_ref, bp2_ref,
                lnps_ref, lnpb_ref, wout_ref, bout_ref, out_ref):
    Q = k_ref.shape[0]
    R = Q * CTX
    k = k_ref[...]
    ki = ki_ref[...]
    rowq = jax.lax.broadcasted_iota(jnp.int32, (R, Q), 0) // CTX
    colq = jax.lax.broadcasted_iota(jnp.int32, (R, Q), 1)
    E = (rowq == colq).astype(jnp.float32)
    kw = jnp.dot(k, wt1_ref[...], preferred_element_type=jnp.float32)
    kiw = jnp.dot(ki, wt1_ref[...], preferred_element_type=jnp.float32)
    kwx = jnp.dot(E, kw, preferred_element_type=jnp.float32)
    h = jnp.maximum(kwx - kiw + bt1_ref[...], 0.0)
    t = jnp.dot(h, wt2_ref[...], preferred_element_type=jnp.float32)
    V = ey_ref[...] + t
    s = ds_ref[...]
    s = s - jnp.max(s, axis=1, keepdims=True)
    es = jnp.exp(s)
    w = es / jnp.sum(es, axis=1, keepdims=True)
    wt = jnp.tile(w, (1, Q))
    colq2 = jax.lax.broadcasted_iota(jnp.int32, (Q, R), 1) // CTX
    rowq2 = jax.lax.broadcasted_iota(jnp.int32, (Q, R), 0)
    wfull = jnp.where(rowq2 == colq2, wt, 0.0)
    Vc = jnp.dot(wfull, V, preferred_element_type=jnp.float32)
    x = xe_ref[...] + Vc
    h1 = _ln_in(x, ln1s_ref[...], ln1b_ref[...])
    h1 = jnp.maximum(
        jnp.dot(h1, wp1_ref[...], preferred_element_type=jnp.float32)
        + bp1_ref[...], 0.0)
    h1 = jnp.dot(h1, wp2_ref[...], preferred_element_type=jnp.float32) + bp2_ref[...]
    x = x + h1
    xo = jnp.maximum(_ln_in(x, lnps_ref[...], lnpb_ref[...]), 0.0)
    out_ref[...] = jnp.dot(xo, wout_ref[...], preferred_element_type=jnp.float32) + bout_ref[...]


def kernel(x_num, cand_x_num, W_lin, b_lin, W_K, b_K, Y_emb, W_T1, b_T1, W_T2,
           ln1_s, ln1_b, W_P1, b_P1, W_P2, b_P2, lnP_s, lnP_b, W_out, b_out,
           cand_y, context_size):
    f32 = jnp.float32
    bl = b_lin.reshape(1, D)
    bk = b_K.reshape(1, D)
    bt1 = b_T1.reshape(1, DI)
    ln1s = ln1_s.reshape(1, D)
    ln1b = ln1_b.reshape(1, D)
    bp1 = b_P1.reshape(1, DI)
    bp2 = b_P2.reshape(1, D)
    lnps = lnP_s.reshape(1, D)
    lnpb = lnP_b.reshape(1, D)
    bout = b_out.reshape(1, NC)

    NT = 2048
    ki_all = pl.pallas_call(
        _enc_cand_body,
        grid=(N // NT,),
        in_specs=[
            pl.BlockSpec((NT, F), lambda i: (i, 0)),
            pl.BlockSpec((F, D), lambda i: (0, 0)),
            pl.BlockSpec((1, D), lambda i: (0, 0)),
            pl.BlockSpec((D, D), lambda i: (0, 0)),
            pl.BlockSpec((1, D), lambda i: (0, 0)),
        ],
        out_specs=pl.BlockSpec((NT, D), lambda i: (i, 0)),
        out_shape=jax.ShapeDtypeStruct((N, D), f32),
    )(cand_x_num, W_lin, bl, W_K, bk)

    BT = 512
    x_enc, k = pl.pallas_call(
        _enc_query_body,
        grid=(B // BT,),
        in_specs=[
            pl.BlockSpec((BT, F), lambda i: (i, 0)),
            pl.BlockSpec((F, D), lambda i: (0, 0)),
            pl.BlockSpec((1, D), lambda i: (0, 0)),
            pl.BlockSpec((D, D), lambda i: (0, 0)),
            pl.BlockSpec((1, D), lambda i: (0, 0)),
        ],
        out_specs=[
            pl.BlockSpec((BT, D), lambda i: (i, 0)),
            pl.BlockSpec((BT, D), lambda i: (i, 0)),
        ],
        out_shape=[
            jax.ShapeDtypeStruct((B, D), f32),
            jax.ShapeDtypeStruct((B, D), f32),
        ],
    )(x_num, W_lin, bl, W_K, bk)

    BQ, NTT = 256, 2048
    dist = pl.pallas_call(
        _dist_body,
        grid=(N // NTT, B // BQ),
        in_specs=[
            pl.BlockSpec((BQ, D), lambda j, i: (i, 0)),
            pl.BlockSpec((NTT, D), lambda j, i: (j, 0)),
        ],
        out_specs=pl.BlockSpec((BQ, NTT), lambda j, i: (i, j)),
        out_shape=jax.ShapeDtypeStruct((B, N), f32),
    )(k, ki_all)

    negd, I = jax.lax.top_k(-dist, CTX)
    dist_sel = -negd
    ki_g = ki_all[I].reshape(B * CTX, D)
    ey = Y_emb[cand_y[I]].reshape(B * CTX, D)

    Q = 16
    out = pl.pallas_call(
        _final_body,
        grid=(B // Q,),
        in_specs=[
            pl.BlockSpec((Q, D), lambda i: (i, 0)),
            pl.BlockSpec((Q, D), lambda i: (i, 0)),
            pl.BlockSpec((Q * CTX, D), lambda i: (i, 0)),
            pl.BlockSpec((Q * CTX, D), lambda i: (i, 0)),
            pl.BlockSpec((Q, CTX), lambda i: (i, 0)),
            pl.BlockSpec((D, DI), lambda i: (0, 0)),
            pl.BlockSpec((1, DI), lambda i: (0, 0)),
            pl.BlockSpec((DI, D), lambda i: (0, 0)),
            pl.BlockSpec((1, D), lambda i: (0, 0)),
            pl.BlockSpec((1, D), lambda i: (0, 0)),
            pl.BlockSpec((D, DI), lambda i: (0, 0)),
            pl.BlockSpec((1, DI), lambda i: (0, 0)),
            pl.BlockSpec((DI, D), lambda i: (0, 0)),
            pl.BlockSpec((1, D), lambda i: (0, 0)),
            pl.BlockSpec((1, D), lambda i: (0, 0)),
            pl.BlockSpec((1, D), lambda i: (0, 0)),
            pl.BlockSpec((D, NC), lambda i: (0, 0)),
            pl.BlockSpec((1, NC), lambda i: (0, 0)),
        ],
        out_specs=pl.BlockSpec((Q, NC), lambda i: (i, 0)),
        out_shape=jax.ShapeDtypeStruct((B, NC), f32),
    )(k, x_enc, ki_g, ey, dist_sel,
      W_T1, bt1, W_T2, ln1s, ln1b, W_P1, bp1, W_P2, bp2,
      lnps, lnpb, W_out, bout)
    return out
